# baseline (speedup 1.0000x reference)
import functools

import jax
import jax.numpy as jnp
from jax import lax
from jax.experimental import pallas as pl
from jax.experimental.pallas import tpu as pltpu
from jax.experimental.pallas import tpu_sc as plsc

N = 10000
D = 128
NPAD = 10240
NC = 2
NS = 16
NTILES = NC * NS
CH = 128
NCHUNK = 80
NBUF = 2
XCH = 8
EPT = NCHUNK * CH
E_PAD = NTILES * EPT
ROWS_PT = NPAD // NS
DUMMY = N
BLK = 1024
GRID = NPAD // BLK


@functools.cache
def _sc_kernels():
    mesh = plsc.VectorSubcoreMesh(core_axis_name="c", subcore_axis_name="s",
                                  num_cores=NC, num_subcores=NS)

    @functools.partial(
        pl.kernel,
        out_type=jax.ShapeDtypeStruct((NC, NPAD, D), jnp.float32),
        mesh=mesh,
        scratch_types=[
            pltpu.VMEM((NCHUNK, CH), jnp.int32),
            pltpu.VMEM((CH, D), jnp.float32),
            pltpu.VMEM_SHARED((NPAD, D), jnp.float32),
            pltpu.SemaphoreType.DMA,
        ],
    )
    def _sc_degree(dst_hbm, ones_hbm, zeros_hbm, out_hbm, dst_v, ones_v, acc_sh,
                   sem):
        c = lax.axis_index("c")
        s = lax.axis_index("s")
        t = c * NS + s
        pltpu.sync_copy(dst_hbm.at[t, pl.ds(0, NCHUNK)], dst_v)
        pltpu.sync_copy(ones_hbm, ones_v)
        r0 = s * ROWS_PT
        pltpu.sync_copy(zeros_hbm.at[pl.ds(r0, ROWS_PT)], acc_sh.at[pl.ds(r0, ROWS_PT)])
        plsc.subcore_barrier()

        @pl.loop(0, NCHUNK)
        def _(j):
            pltpu.async_copy(ones_v, acc_sh.at[dst_v.at[j]], sem, add=True)

        @pl.loop(0, NCHUNK)
        def _(j):
            pltpu.make_async_copy(ones_v, acc_sh.at[pl.ds(0, CH)], sem).wait()

        plsc.subcore_barrier()
        pltpu.sync_copy(acc_sh.at[pl.ds(r0, ROWS_PT)], out_hbm.at[c, pl.ds(r0, ROWS_PT)])

    PCH = NCHUNK // 2

    @functools.partial(
        pl.kernel,
        out_type=jax.ShapeDtypeStruct((NC, NPAD, D), jnp.float32),
        mesh=mesh,
        scratch_types=[
            pltpu.VMEM((PCH + XCH, CH), jnp.int32),
            pltpu.VMEM((PCH, CH), jnp.int32),
            pltpu.VMEM_SHARED((NPAD, D), jnp.float32),
        ]
        + [pltpu.VMEM((CH, D), jnp.float32)] * NBUF
        + [pltpu.SemaphoreType.DMA] * NBUF,
    )
    def _sc_aggregate(y_hbm, src_hbm, dst_hbm, zeros_hbm, out_hbm,
                      src_v, dst_v, acc_sh, *scr):
        rows = scr[:NBUF]
        gsem = scr[NBUF:]
        c = lax.axis_index("c")
        s = lax.axis_index("s")
        t = c * NS + s
        r0 = s * ROWS_PT
        pltpu.sync_copy(zeros_hbm.at[pl.ds(r0, ROWS_PT)], acc_sh.at[pl.ds(r0, ROWS_PT)])
        plsc.subcore_barrier()

        def wait_rows(sem):
            pltpu.make_async_copy(y_hbm.at[pl.ds(0, CH)], rows[0], sem).wait()

        for p in range(2):
            base = p * PCH
            pltpu.sync_copy(src_hbm.at[t, pl.ds(base, PCH + XCH)], src_v)
            pltpu.sync_copy(dst_hbm.at[t, pl.ds(base, PCH)], dst_v)
            pltpu.async_copy(y_hbm.at[src_v.at[0]], rows[0], gsem[0])

            @pl.loop(0, PCH, step=NBUF)
            def _(j0):
                for b in range(NBUF):
                    j = j0 + b
                    wait_rows(gsem[b])
                    pltpu.async_copy(y_hbm.at[src_v.at[j + 1]], rows[1 - b],
                                     gsem[1 - b])
                    pltpu.sync_copy(rows[b], acc_sh.at[dst_v.at[j]], add=True)

            wait_rows(gsem[0])

        plsc.subcore_barrier()
        pltpu.sync_copy(acc_sh.at[pl.ds(r0, ROWS_PT)], out_hbm.at[c, pl.ds(r0, ROWS_PT)])

    return _sc_degree, _sc_aggregate


def _tc_scale_kernel(x_ref, degp_ref, w_ref, y_ref, dinv_ref):
    p = degp_ref[0] + degp_ref[1]
    deg = p[:, 0:1] + 1.0
    dinv = lax.rsqrt(deg)
    dinvb = jnp.broadcast_to(dinv, (BLK, D))
    xw = jnp.dot(x_ref[...], w_ref[...], precision=lax.Precision.HIGHEST,
                 preferred_element_type=jnp.float32)
    y_ref[...] = dinvb * xw
    dinv_ref[...] = dinvb


def _tc_layer_kernel(sp_ref, y_ref, dinv_ref, b_ref, w_ref, y2_ref):
    agg = sp_ref[0] + sp_ref[1] + y_ref[...]
    h = jnp.maximum(dinv_ref[...] * agg + b_ref[...], 0.0)
    xw = jnp.dot(h, w_ref[...], precision=lax.Precision.HIGHEST,
                 preferred_element_type=jnp.float32)
    y2_ref[...] = dinv_ref[...] * xw


def _tc_pool_kernel(sp_ref, y_ref, dinv_ref, b_ref, out_ref):
    i = pl.program_id(0)
    agg = sp_ref[0] + sp_ref[1] + y_ref[...]
    h = jnp.maximum(dinv_ref[...] * agg + b_ref[...], 0.0)
    row = lax.broadcasted_iota(jnp.int32, (BLK, D), 0) + i * BLK
    h = jnp.where(row < N, h, 0.0)
    part = jnp.sum(h, axis=0, keepdims=True) * (1.0 / N)

    @pl.when(i == 0)
    def _():
        out_ref[...] = jnp.zeros_like(out_ref)

    out_ref[...] += part


_row_spec = pl.BlockSpec((BLK, D), lambda i: (i, 0))
_pair_spec = pl.BlockSpec((NC, BLK, D), lambda i: (0, i, 0))
_w_spec = pl.BlockSpec((D, D), lambda i: (0, 0))
_b_spec = pl.BlockSpec((1, D), lambda i: (0, 0))

_tc_scale = pl.pallas_call(
    _tc_scale_kernel,
    grid=(GRID,),
    in_specs=[_row_spec, _pair_spec, _w_spec],
    out_specs=[_row_spec, _row_spec],
    out_shape=[jax.ShapeDtypeStruct((NPAD, D), jnp.float32),
               jax.ShapeDtypeStruct((NPAD, D), jnp.float32)],
)

_tc_layer = pl.pallas_call(
    _tc_layer_kernel,
    grid=(GRID,),
    in_specs=[_pair_spec, _row_spec, _row_spec, _b_spec, _w_spec],
    out_specs=_row_spec,
    out_shape=jax.ShapeDtypeStruct((NPAD, D), jnp.float32),
)

_tc_pool = pl.pallas_call(
    _tc_pool_kernel,
    grid=(GRID,),
    in_specs=[_pair_spec, _row_spec, _row_spec, _b_spec],
    out_specs=pl.BlockSpec((1, D), lambda i: (0, 0)),
    out_shape=jax.ShapeDtypeStruct((1, D), jnp.float32),
)


def kernel(x, edge_index, W1, b1, W2, b2):
    src = edge_index[0].astype(jnp.int32)
    dst = edge_index[1].astype(jnp.int32)
    npad_e = E_PAD - src.shape[0]
    src_t = jnp.concatenate(
        [src, jnp.zeros((npad_e,), jnp.int32)]).reshape(NTILES, NCHUNK, CH)
    src_t = jnp.concatenate(
        [src_t, jnp.zeros((NTILES, XCH, CH), jnp.int32)], axis=1)
    pad_dst = DUMMY + jnp.arange(npad_e, dtype=jnp.int32) % (NPAD - N)
    dst_t = jnp.concatenate([dst, pad_dst]).reshape(NTILES, NCHUNK, CH)
    dst_t = jnp.concatenate(
        [dst_t, jnp.full((NTILES, NBUF, CH), DUMMY, jnp.int32)], axis=1)

    xp = jnp.pad(x, ((0, NPAD - N), (0, 0)))
    onesD = jnp.ones((CH, D), jnp.float32)
    zerosD = jnp.zeros((NPAD, D), jnp.float32)
    b1r = b1.reshape(1, D)
    b2r = b2.reshape(1, D)

    sc_degree, sc_aggregate = _sc_kernels()
    degp = sc_degree(dst_t, onesD, zerosD)
    y1, dinvb = _tc_scale(xp, degp, W1)
    s1p = sc_aggregate(y1, src_t, dst_t, zerosD)
    y2 = _tc_layer(s1p, y1, dinvb, b1r, W2)
    s2p = sc_aggregate(y2, src_t, dst_t, zerosD)
    return _tc_pool(s2p, y2, dinvb, b2r)


# speedup vs baseline: 1.1915x; 1.1915x over previous
import functools

import jax
import jax.numpy as jnp
from jax import lax
from jax.experimental import pallas as pl
from jax.experimental.pallas import tpu as pltpu
from jax.experimental.pallas import tpu_sc as plsc

N = 10000
D = 128
NPAD = 10240
NC = 2
NS = 16
NTILES = NC * NS
CH = 128
NCHUNK = 80
NBUF = 2
XCH = 8
EPT = NCHUNK * CH
E_PAD = NTILES * EPT
ROWS_PT = NPAD // NS
DUMMY = N
BLK = 1024
GRID = NPAD // BLK


@functools.cache
def _sc_kernels():
    mesh = plsc.VectorSubcoreMesh(core_axis_name="c", subcore_axis_name="s",
                                  num_cores=NC, num_subcores=NS)

    @functools.partial(
        pl.kernel,
        out_type=jax.ShapeDtypeStruct((NC, NPAD, D), jnp.float32),
        mesh=mesh,
        scratch_types=[
            pltpu.VMEM((NCHUNK, CH), jnp.int32),
            pltpu.VMEM((CH, D), jnp.float32),
            pltpu.VMEM_SHARED((NPAD, D), jnp.float32),
            pltpu.SemaphoreType.DMA,
        ],
    )
    def _sc_degree(dst_hbm, ones_hbm, zeros_hbm, out_hbm, dst_v, ones_v, acc_sh,
                   sem):
        c = lax.axis_index("c")
        s = lax.axis_index("s")
        t = c * NS + s
        pltpu.sync_copy(dst_hbm.at[t, pl.ds(0, NCHUNK)], dst_v)
        pltpu.sync_copy(ones_hbm, ones_v)
        r0 = s * ROWS_PT
        pltpu.sync_copy(zeros_hbm.at[pl.ds(r0, ROWS_PT)], acc_sh.at[pl.ds(r0, ROWS_PT)])
        plsc.subcore_barrier()

        @pl.loop(0, NCHUNK)
        def _(j):
            pltpu.async_copy(ones_v, acc_sh.at[dst_v.at[j]], sem, add=True)

        @pl.loop(0, NCHUNK)
        def _(j):
            pltpu.make_async_copy(ones_v, acc_sh.at[pl.ds(0, CH)], sem).wait()

        plsc.subcore_barrier()
        pltpu.sync_copy(acc_sh.at[pl.ds(r0, ROWS_PT)], out_hbm.at[c, pl.ds(r0, ROWS_PT)])

    @functools.partial(
        pl.kernel,
        out_type=jax.ShapeDtypeStruct((NC, NPAD, D), jnp.float32),
        mesh=mesh,
        scratch_types=[
            pltpu.VMEM((NCHUNK, CH), jnp.int32),
            pltpu.VMEM((NCHUNK, CH), jnp.int32),
            pltpu.VMEM((CH, D), jnp.float32),
            pltpu.VMEM_SHARED((NPAD, D), jnp.float32),
            pltpu.SemaphoreType.DMA,
        ],
    )
    def _sc_aggregate(y_hbm, src_hbm, dst_hbm, zeros_hbm, out_hbm,
                      src_v, dst_v, rows_v, acc_sh, sem):
        c = lax.axis_index("c")
        s = lax.axis_index("s")
        t = c * NS + s
        pltpu.sync_copy(src_hbm.at[t, pl.ds(0, NCHUNK)], src_v)
        pltpu.sync_copy(dst_hbm.at[t, pl.ds(0, NCHUNK)], dst_v)
        r0 = s * ROWS_PT
        pltpu.sync_copy(zeros_hbm.at[pl.ds(r0, ROWS_PT)], acc_sh.at[pl.ds(r0, ROWS_PT)])
        plsc.subcore_barrier()

        @pl.loop(0, NCHUNK)
        def _(j):
            pltpu.async_copy(y_hbm.at[src_v.at[j]], rows_v, sem).wait()
            pltpu.sync_copy(rows_v, acc_sh.at[dst_v.at[j]], add=True)

        plsc.subcore_barrier()
        pltpu.sync_copy(acc_sh.at[pl.ds(r0, ROWS_PT)], out_hbm.at[c, pl.ds(r0, ROWS_PT)])

    return _sc_degree, _sc_aggregate


def _tc_scale_kernel(x_ref, degp_ref, w_ref, y_ref, dinv_ref):
    p = degp_ref[0] + degp_ref[1]
    deg = p[:, 0:1] + 1.0
    dinv = lax.rsqrt(deg)
    dinvb = jnp.broadcast_to(dinv, (BLK, D))
    xw = jnp.dot(x_ref[...], w_ref[...], precision=lax.Precision.HIGHEST,
                 preferred_element_type=jnp.float32)
    y_ref[...] = dinvb * xw
    dinv_ref[...] = dinvb


def _tc_layer_kernel(sp_ref, y_ref, dinv_ref, b_ref, w_ref, y2_ref):
    agg = sp_ref[0] + sp_ref[1] + y_ref[...]
    h = jnp.maximum(dinv_ref[...] * agg + b_ref[...], 0.0)
    xw = jnp.dot(h, w_ref[...], precision=lax.Precision.HIGHEST,
                 preferred_element_type=jnp.float32)
    y2_ref[...] = dinv_ref[...] * xw


def _tc_pool_kernel(sp_ref, y_ref, dinv_ref, b_ref, out_ref):
    i = pl.program_id(0)
    agg = sp_ref[0] + sp_ref[1] + y_ref[...]
    h = jnp.maximum(dinv_ref[...] * agg + b_ref[...], 0.0)
    row = lax.broadcasted_iota(jnp.int32, (BLK, D), 0) + i * BLK
    h = jnp.where(row < N, h, 0.0)
    part = jnp.sum(h, axis=0, keepdims=True) * (1.0 / N)

    @pl.when(i == 0)
    def _():
        out_ref[...] = jnp.zeros_like(out_ref)

    out_ref[...] += part


_row_spec = pl.BlockSpec((BLK, D), lambda i: (i, 0))
_pair_spec = pl.BlockSpec((NC, BLK, D), lambda i: (0, i, 0))
_w_spec = pl.BlockSpec((D, D), lambda i: (0, 0))
_b_spec = pl.BlockSpec((1, D), lambda i: (0, 0))

_tc_scale = pl.pallas_call(
    _tc_scale_kernel,
    grid=(GRID,),
    in_specs=[_row_spec, _pair_spec, _w_spec],
    out_specs=[_row_spec, _row_spec],
    out_shape=[jax.ShapeDtypeStruct((NPAD, D), jnp.float32),
               jax.ShapeDtypeStruct((NPAD, D), jnp.float32)],
)

_tc_layer = pl.pallas_call(
    _tc_layer_kernel,
    grid=(GRID,),
    in_specs=[_pair_spec, _row_spec, _row_spec, _b_spec, _w_spec],
    out_specs=_row_spec,
    out_shape=jax.ShapeDtypeStruct((NPAD, D), jnp.float32),
)

_tc_pool = pl.pallas_call(
    _tc_pool_kernel,
    grid=(GRID,),
    in_specs=[_pair_spec, _row_spec, _row_spec, _b_spec],
    out_specs=pl.BlockSpec((1, D), lambda i: (0, 0)),
    out_shape=jax.ShapeDtypeStruct((1, D), jnp.float32),
)


def kernel(x, edge_index, W1, b1, W2, b2):
    src = edge_index[0].astype(jnp.int32)
    dst = edge_index[1].astype(jnp.int32)
    npad_e = E_PAD - src.shape[0]
    src_t = jnp.concatenate(
        [src, jnp.zeros((npad_e,), jnp.int32)]).reshape(NTILES, NCHUNK, CH)
    src_t = jnp.concatenate(
        [src_t, jnp.zeros((NTILES, XCH, CH), jnp.int32)], axis=1)
    pad_dst = DUMMY + jnp.arange(npad_e, dtype=jnp.int32) % (NPAD - N)
    dst_t = jnp.concatenate([dst, pad_dst]).reshape(NTILES, NCHUNK, CH)
    dst_t = jnp.concatenate(
        [dst_t, jnp.full((NTILES, NBUF, CH), DUMMY, jnp.int32)], axis=1)

    xp = jnp.pad(x, ((0, NPAD - N), (0, 0)))
    onesD = jnp.ones((CH, D), jnp.float32)
    zerosD = jnp.zeros((NPAD, D), jnp.float32)
    b1r = b1.reshape(1, D)
    b2r = b2.reshape(1, D)

    sc_degree, sc_aggregate = _sc_kernels()
    degp = sc_degree(dst_t, onesD, zerosD)
    y1, dinvb = _tc_scale(xp, degp, W1)
    s1p = sc_aggregate(y1, src_t, dst_t, zerosD)
    y2 = _tc_layer(s1p, y1, dinvb, b1r, W2)
    s2p = sc_aggregate(y2, src_t, dst_t, zerosD)
    return _tc_pool(s2p, y2, dinvb, b2r)


# speedup vs baseline: 1.2726x; 1.0680x over previous
import functools

import jax
import jax.numpy as jnp
from jax import lax
from jax.experimental import pallas as pl
from jax.experimental.pallas import tpu as pltpu
from jax.experimental.pallas import tpu_sc as plsc

N = 10000
D = 128
NPAD = 10240
NC = 2
NS = 16
NTILES = NC * NS
CH = 128
NCHUNK = 80
EPT = NCHUNK * CH
E_PAD = NTILES * EPT
ROWS_PT = NPAD // NS
DUMMY = N
BLK = 1024
GRID = NPAD // BLK


@functools.cache
def _sc_kernels():
    mesh = plsc.VectorSubcoreMesh(core_axis_name="c", subcore_axis_name="s",
                                  num_cores=NC, num_subcores=NS)

    @functools.partial(
        pl.kernel,
        out_type=jax.ShapeDtypeStruct((NC, NPAD, D), jnp.float32),
        mesh=mesh,
        scratch_types=[
            pltpu.VMEM((NCHUNK, CH), jnp.int32),
            pltpu.VMEM((CH, D), jnp.float32),
            pltpu.VMEM_SHARED((NPAD, D), jnp.float32),
            pltpu.SemaphoreType.DMA,
        ],
    )
    def _sc_degree(dst_hbm, ones_hbm, zeros_hbm, out_hbm, dst_v, ones_v, acc_sh,
                   sem):
        c = lax.axis_index("c")
        s = lax.axis_index("s")
        t = c * NS + s
        pltpu.sync_copy(dst_hbm.at[t], dst_v)
        pltpu.sync_copy(ones_hbm, ones_v)
        r0 = s * ROWS_PT
        pltpu.sync_copy(zeros_hbm.at[pl.ds(r0, ROWS_PT)], acc_sh.at[pl.ds(r0, ROWS_PT)])
        plsc.subcore_barrier()

        @pl.loop(0, NCHUNK)
        def _(j):
            pltpu.async_copy(ones_v, acc_sh.at[dst_v.at[j]], sem, add=True)

        @pl.loop(0, NCHUNK)
        def _(j):
            pltpu.make_async_copy(ones_v, acc_sh.at[pl.ds(0, CH)], sem).wait()

        plsc.subcore_barrier()
        pltpu.sync_copy(acc_sh.at[pl.ds(r0, ROWS_PT)], out_hbm.at[c, pl.ds(r0, ROWS_PT)])

    @functools.partial(
        pl.kernel,
        out_type=jax.ShapeDtypeStruct((NC, NPAD, D), jnp.float32),
        mesh=mesh,
        scratch_types=[
            pltpu.VMEM((NCHUNK, CH), jnp.int32),
            pltpu.VMEM((NCHUNK, CH), jnp.int32),
            pltpu.VMEM((CH, D), jnp.float32),
            pltpu.VMEM_SHARED((NPAD, D), jnp.float32),
            pltpu.SemaphoreType.DMA,
        ],
    )
    def _sc_aggregate(y_hbm, src_hbm, dst_hbm, zeros_hbm, out_hbm,
                      src_v, dst_v, rows_v, acc_sh, sem):
        c = lax.axis_index("c")
        s = lax.axis_index("s")
        t = c * NS + s
        pltpu.sync_copy(src_hbm.at[t], src_v)
        pltpu.sync_copy(dst_hbm.at[t], dst_v)
        r0 = s * ROWS_PT
        pltpu.sync_copy(zeros_hbm.at[pl.ds(r0, ROWS_PT)], acc_sh.at[pl.ds(r0, ROWS_PT)])
        plsc.subcore_barrier()

        @pl.loop(0, NCHUNK)
        def _(j):
            pltpu.async_copy(y_hbm.at[src_v.at[j]], rows_v, sem).wait()
            pltpu.sync_copy(rows_v, acc_sh.at[dst_v.at[j]], add=True)

        plsc.subcore_barrier()
        pltpu.sync_copy(acc_sh.at[pl.ds(r0, ROWS_PT)], out_hbm.at[c, pl.ds(r0, ROWS_PT)])

    return _sc_degree, _sc_aggregate


def _tc_scale_kernel(x_ref, degp_ref, w_ref, y_ref, dinv_ref):
    p = degp_ref[0] + degp_ref[1]
    deg = p[:, 0:1] + 1.0
    dinv = lax.rsqrt(deg)
    dinvb = jnp.broadcast_to(dinv, (BLK, D))
    xw = jnp.dot(x_ref[...], w_ref[...], precision=lax.Precision.HIGHEST,
                 preferred_element_type=jnp.float32)
    y_ref[...] = dinvb * xw
    dinv_ref[...] = dinvb


def _tc_layer_kernel(sp_ref, y_ref, dinv_ref, b_ref, w_ref, y2_ref):
    agg = sp_ref[0] + sp_ref[1] + y_ref[...]
    h = jnp.maximum(dinv_ref[...] * agg + b_ref[...], 0.0)
    xw = jnp.dot(h, w_ref[...], precision=lax.Precision.HIGHEST,
                 preferred_element_type=jnp.float32)
    y2_ref[...] = dinv_ref[...] * xw


def _tc_pool_kernel(sp_ref, y_ref, dinv_ref, b_ref, out_ref):
    i = pl.program_id(0)
    agg = sp_ref[0] + sp_ref[1] + y_ref[...]
    h = jnp.maximum(dinv_ref[...] * agg + b_ref[...], 0.0)
    row = lax.broadcasted_iota(jnp.int32, (BLK, D), 0) + i * BLK
    h = jnp.where(row < N, h, 0.0)
    part = jnp.sum(h, axis=0, keepdims=True) * (1.0 / N)

    @pl.when(i == 0)
    def _():
        out_ref[...] = jnp.zeros_like(out_ref)

    out_ref[...] += part


_row_spec = pl.BlockSpec((BLK, D), lambda i: (i, 0))
_pair_spec = pl.BlockSpec((NC, BLK, D), lambda i: (0, i, 0))
_w_spec = pl.BlockSpec((D, D), lambda i: (0, 0))
_b_spec = pl.BlockSpec((1, D), lambda i: (0, 0))

_tc_scale = pl.pallas_call(
    _tc_scale_kernel,
    grid=(GRID,),
    in_specs=[_row_spec, _pair_spec, _w_spec],
    out_specs=[_row_spec, _row_spec],
    out_shape=[jax.ShapeDtypeStruct((NPAD, D), jnp.float32),
               jax.ShapeDtypeStruct((NPAD, D), jnp.float32)],
)

_tc_layer = pl.pallas_call(
    _tc_layer_kernel,
    grid=(GRID,),
    in_specs=[_pair_spec, _row_spec, _row_spec, _b_spec, _w_spec],
    out_specs=_row_spec,
    out_shape=jax.ShapeDtypeStruct((NPAD, D), jnp.float32),
)

_tc_pool = pl.pallas_call(
    _tc_pool_kernel,
    grid=(GRID,),
    in_specs=[_pair_spec, _row_spec, _row_spec, _b_spec],
    out_specs=pl.BlockSpec((1, D), lambda i: (0, 0)),
    out_shape=jax.ShapeDtypeStruct((1, D), jnp.float32),
)


def kernel(x, edge_index, W1, b1, W2, b2):
    src = edge_index[0].astype(jnp.int32)
    dst = edge_index[1].astype(jnp.int32)
    npad_e = E_PAD - src.shape[0]
    src_t = jnp.concatenate(
        [src, jnp.zeros((npad_e,), jnp.int32)]).reshape(NTILES, NCHUNK, CH)
    pad_dst = DUMMY + jnp.arange(npad_e, dtype=jnp.int32) % (NPAD - N)
    dst_t = jnp.concatenate([dst, pad_dst]).reshape(NTILES, NCHUNK, CH)

    xp = jnp.pad(x, ((0, NPAD - N), (0, 0)))
    onesD = jnp.ones((CH, D), jnp.float32)
    zerosD = jnp.zeros((NPAD, D), jnp.float32)
    b1r = b1.reshape(1, D)
    b2r = b2.reshape(1, D)

    sc_degree, sc_aggregate = _sc_kernels()
    degp = sc_degree(dst_t, onesD, zerosD)
    y1, dinvb = _tc_scale(xp, degp, W1)
    s1p = sc_aggregate(y1, src_t, dst_t, zerosD)
    y2 = _tc_layer(s1p, y1, dinvb, b1r, W2)
    s2p = sc_aggregate(y2, src_t, dst_t, zerosD)
    return _tc_pool(s2p, y2, dinvb, b2r)
